# baseline (speedup 1.0000x reference)
import functools

import jax
import jax.numpy as jnp
from jax import lax
from jax.experimental import pallas as pl
from jax.experimental.pallas import tpu as pltpu
from jax.experimental.pallas import tpu_sc as plsc

N = 10000
E = 320000
D = 128
G = 64
WROW = 144
NPAD = 10000
NTILES = 32
CH = 128
NCHUNK = 80
EPT = NCHUNK * CH
EPAD = EPT * NTILES
BN = 1000
NBLK = N // BN


def _edge_body(hpad_hbm, ad_hbm, pk_hbm, out_hbm,
               ad_t, rg, pkb, scur, dgc, w_v,
               acc_sh, gsem):
    c = lax.axis_index("c")
    s = lax.axis_index("s")
    wid = s * 2 + c
    base_e = wid * EPT

    pltpu.sync_copy(ad_hbm, ad_t.at[pl.ds(0, N)])
    ad_t[pl.ds(N, 16)] = jnp.zeros((16,), jnp.float32)

    for b in range(16):
        for j in range(WROW // 16):
            rg[b, pl.ds(j * 16, 16)] = jnp.zeros((16,), jnp.float32)

    def zstep(k, _):
        pltpu.sync_copy(rg.at[pl.ds(0, 16)],
                        acc_sh.at[pl.ds(s * 625 + k * 16, 16)])
        return 0
    lax.fori_loop(0, 39, zstep, 0)
    pltpu.sync_copy(rg.at[pl.ds(0, 1)], acc_sh.at[pl.ds(s * 625 + 624, 1)])
    plsc.subcore_barrier()

    def chunk(i, _):
        pltpu.sync_copy(pk_hbm.at[pl.ds(base_e + i * CH, CH)], pkb)
        for j in range(CH // 16):
            pk = pkb[pl.ds(j * 16, 16)]
            scur[pl.ds(j * 16, 16)] = jnp.bitwise_and(pk, 16383)
            dgc[pl.ds(j * 16, 16)] = jnp.right_shift(pk, 14)
        pltpu.async_copy(hpad_hbm.at[scur], rg, gsem).wait()
        for j in range(CH // 16):
            rows16 = jnp.arange(16, dtype=jnp.int32) + j * 16
            asv = plsc.load_gather(rg, [rows16,
                                        jnp.full((16,), 129, jnp.int32)])
            dv = dgc[pl.ds(j * 16, 16)]
            e = asv + plsc.load_gather(ad_t, [dv])
            e = jnp.maximum(e, e * 0.2)
            w_v[pl.ds(j * 16, 16)] = jnp.exp(e)

        def grp(gi, _):
            wv = w_v[pl.ds(gi * 16, 16)]
            for l in range(16):
                wl = wv[l]
                b = gi * 16 + l
                for j in range(WROW // 16):
                    rg[b, pl.ds(j * 16, 16)] = rg[b, pl.ds(j * 16, 16)] * wl
            return 0
        lax.fori_loop(0, CH // 16, grp, 0)
        pltpu.sync_copy(rg, acc_sh.at[dgc], add=True)
        return 0
    lax.fori_loop(0, NCHUNK, chunk, 0)
    plsc.subcore_barrier()

    r0 = s * 625
    pltpu.sync_copy(acc_sh.at[pl.ds(r0, 625)], out_hbm.at[c, pl.ds(r0, 625)])


_edge_pass = functools.partial(
    pl.kernel,
    out_type=jax.ShapeDtypeStruct((2, N, WROW), jnp.float32),
    mesh=plsc.VectorSubcoreMesh(core_axis_name="c", subcore_axis_name="s"),
    compiler_params=pltpu.CompilerParams(
        needs_layout_passes=False, use_tc_tiling_on_sc=False),
    scratch_types=[
        pltpu.VMEM((N + 16,), jnp.float32),
        pltpu.VMEM((CH, WROW), jnp.float32),
        pltpu.VMEM((CH,), jnp.int32),
        pltpu.VMEM((CH,), jnp.int32),
        pltpu.VMEM((CH,), jnp.int32),
        pltpu.VMEM((CH,), jnp.float32),
        pltpu.VMEM_SHARED((NPAD, WROW), jnp.float32),
        pltpu.SemaphoreType.DMA,
    ],
)(_edge_body)


def _emit_layer_outputs(h, as_v, ad_ref, ad_v, hpad_ref):
    hpad_ref[:, :D] = h
    hpad_ref[:, D:D + 1] = jnp.ones((BN, 1), jnp.float32)
    hpad_ref[:, D + 1:D + 2] = as_v
    hpad_ref[:, D + 2:] = jnp.zeros((BN, WROW - D - 2), jnp.float32)
    ad_ref[...] = ad_v


def _tc_a_body(x_ref, w_ref, avs_ref, avd_ref, hpad_ref, ad_ref):
    h = jnp.dot(x_ref[...], w_ref[...], preferred_element_type=jnp.float32)
    as_v = jnp.dot(h, avs_ref[...], preferred_element_type=jnp.float32)
    ad_v = jnp.dot(h, avd_ref[...], preferred_element_type=jnp.float32)
    _emit_layer_outputs(h, as_v, ad_ref, ad_v, hpad_ref)


def _tc_a(x, w, avs, avd):
    return pl.pallas_call(
        _tc_a_body,
        grid=(NBLK,),
        in_specs=[
            pl.BlockSpec((BN, D), lambda i: (i, 0)),
            pl.BlockSpec((D, D), lambda i: (0, 0)),
            pl.BlockSpec((D, 1), lambda i: (0, 0)),
            pl.BlockSpec((D, 1), lambda i: (0, 0)),
        ],
        out_specs=[
            pl.BlockSpec((BN, WROW), lambda i: (i, 0)),
            pl.BlockSpec((BN, 1), lambda i: (i, 0)),
        ],
        out_shape=[
            jax.ShapeDtypeStruct((N, WROW), jnp.float32),
            jax.ShapeDtypeStruct((N, 1), jnp.float32),
        ],
    )(x, w, avs, avd)


def _combine(part_ref, b_ref):
    p0 = part_ref[0]
    p1 = part_ref[1]
    den = p0[:, D:D + 1] + p1[:, D:D + 1] + 1e-16
    return (p0[:, :D] + p1[:, :D]) / den + b_ref[...]


def _tc_b_body(part_ref, b_ref, w_ref, avs_ref, avd_ref, hpad_ref, ad_ref):
    feats = _combine(part_ref, b_ref)
    h = jnp.dot(feats, w_ref[...], preferred_element_type=jnp.float32)
    as_v = jnp.dot(h, avs_ref[...], preferred_element_type=jnp.float32)
    ad_v = jnp.dot(h, avd_ref[...], preferred_element_type=jnp.float32)
    _emit_layer_outputs(h, as_v, ad_ref, ad_v, hpad_ref)


def _tc_b(part, b, w, avs, avd):
    return pl.pallas_call(
        _tc_b_body,
        grid=(NBLK,),
        in_specs=[
            pl.BlockSpec((2, BN, WROW), lambda i: (0, i, 0)),
            pl.BlockSpec((1, D), lambda i: (0, 0)),
            pl.BlockSpec((D, D), lambda i: (0, 0)),
            pl.BlockSpec((D, 1), lambda i: (0, 0)),
            pl.BlockSpec((D, 1), lambda i: (0, 0)),
        ],
        out_specs=[
            pl.BlockSpec((BN, WROW), lambda i: (i, 0)),
            pl.BlockSpec((BN, 1), lambda i: (i, 0)),
        ],
        out_shape=[
            jax.ShapeDtypeStruct((N, WROW), jnp.float32),
            jax.ShapeDtypeStruct((N, 1), jnp.float32),
        ],
    )(part, b, w, avs, avd)


def _tc_c_body(part_ref, b_ref, batch_ref, out_ref, sums, cnt):
    i = pl.program_id(0)

    @pl.when(i == 0)
    def _():
        sums[...] = jnp.zeros_like(sums)
        cnt[...] = jnp.zeros_like(cnt)

    feats = _combine(part_ref, b_ref)
    bblk = batch_ref[0, 0, :]
    oh = (bblk[None, :] == lax.broadcasted_iota(jnp.int32, (G, BN), 0))
    oh = oh.astype(jnp.float32)
    sums[...] += jnp.dot(oh, feats, preferred_element_type=jnp.float32)
    cnt[...] += jnp.sum(oh, axis=1, keepdims=True)

    @pl.when(i == NBLK - 1)
    def _():
        out_ref[...] = sums[...] / jnp.maximum(cnt[...], 1.0)


def _tc_c(part, b, batch3):
    return pl.pallas_call(
        _tc_c_body,
        grid=(NBLK,),
        in_specs=[
            pl.BlockSpec((2, BN, WROW), lambda i: (0, i, 0)),
            pl.BlockSpec((1, D), lambda i: (0, 0)),
            pl.BlockSpec((1, 1, BN), lambda i: (i, 0, 0)),
        ],
        out_specs=pl.BlockSpec((G, D), lambda i: (0, 0)),
        out_shape=jax.ShapeDtypeStruct((G, D), jnp.float32),
        scratch_shapes=[
            pltpu.VMEM((G, D), jnp.float32),
            pltpu.VMEM((G, 1), jnp.float32),
        ],
    )(part, b, batch3)


def kernel(x, edge_index, batch, W1, att_src1, att_dst1, b1,
           W2, att_src2, att_dst2, b2):
    srcp = jnp.concatenate(
        [edge_index[0], jnp.full((EPAD - E,), N, jnp.int32)])
    dstp = jnp.concatenate(
        [edge_index[1],
         jnp.arange(EPAD - E, dtype=jnp.int32) * 13 % N])
    packed = jnp.bitwise_or(srcp, jnp.left_shift(dstp, 14))
    zrow = jnp.zeros((8, WROW), jnp.float32)

    hpad1, ad1 = _tc_a(x, W1, att_src1.reshape(D, 1), att_dst1.reshape(D, 1))
    part1 = _edge_pass(jnp.concatenate([hpad1, zrow]), ad1.reshape(N), packed)
    hpad2, ad2 = _tc_b(part1, b1.reshape(1, D), W2,
                       att_src2.reshape(D, 1), att_dst2.reshape(D, 1))
    part2 = _edge_pass(jnp.concatenate([hpad2, zrow]), ad2.reshape(N), packed)
    return _tc_c(part2, b2.reshape(1, D), batch.reshape(NBLK, 1, BN))


# speedup vs baseline: 1.2960x; 1.2960x over previous
import functools

import jax
import jax.numpy as jnp
from jax import lax
from jax.experimental import pallas as pl
from jax.experimental.pallas import tpu as pltpu
from jax.experimental.pallas import tpu_sc as plsc

N = 10000
E = 320000
D = 128
G = 64
WROW = 144
NPAD = 10240
NTILES = 32
EPT = 10112
EPAD = EPT * NTILES
CH = 128
NCHUNK = EPT // CH
BN = 1000
NBLK = N // BN


def _edge_body(hpad_hbm, as_hbm, ad_hbm, src_hbm, dst_hbm, out_hbm,
               src_v, dst_v, w_v, as_t, ad_t, rows_v, acc_sh, sem):
    c = lax.axis_index("c")
    s = lax.axis_index("s")
    wid = s * 2 + c

    pltpu.sync_copy(as_hbm, as_t)
    pltpu.sync_copy(ad_hbm, ad_t.at[pl.ds(0, N)])
    ad_t[pl.ds(N, 16)] = jnp.zeros((16,), jnp.float32)

    for b in range(16):
        for j in range(WROW // 16):
            rows_v[b, pl.ds(j * 16, 16)] = jnp.zeros((16,), jnp.float32)

    def zero_step(k, _):
        pltpu.sync_copy(rows_v.at[pl.ds(0, 16)],
                        acc_sh.at[pl.ds(s * 640 + k * 16, 16)])
        return 0
    lax.fori_loop(0, 40, zero_step, 0)
    plsc.subcore_barrier()

    base_e = wid * EPT

    def chunk(i, _):
        off = base_e + i * CH
        pltpu.sync_copy(src_hbm.at[pl.ds(off, CH)], src_v)
        pltpu.sync_copy(dst_hbm.at[pl.ds(off, CH)], dst_v)
        for j in range(CH // 16):
            sv = src_v[pl.ds(j * 16, 16)]
            dv = dst_v[pl.ds(j * 16, 16)]
            e = plsc.load_gather(as_t, [sv]) + plsc.load_gather(ad_t, [dv])
            e = jnp.maximum(e, e * 0.2)
            w_v[pl.ds(j * 16, 16)] = jnp.exp(e)
        pltpu.async_copy(hpad_hbm.at[src_v], rows_v, sem).wait()
        def scale(g, _):
            wv = w_v[pl.ds(g * 16, 16)]
            for l in range(16):
                wb = wv[l]
                b = g * 16 + l
                for j in range(WROW // 16):
                    rows_v[b, pl.ds(j * 16, 16)] = (
                        rows_v[b, pl.ds(j * 16, 16)] * wb)
            return 0
        lax.fori_loop(0, CH // 16, scale, 0)
        pltpu.sync_copy(rows_v, acc_sh.at[dst_v], add=True)
        return 0
    lax.fori_loop(0, NCHUNK, chunk, 0)
    plsc.subcore_barrier()

    r0 = s * 624
    pltpu.sync_copy(acc_sh.at[pl.ds(r0, 640)], out_hbm.at[c, pl.ds(r0, 640)])


_edge_pass = functools.partial(
    pl.kernel,
    out_type=jax.ShapeDtypeStruct((2, N, WROW), jnp.float32),
    mesh=plsc.VectorSubcoreMesh(core_axis_name="c", subcore_axis_name="s"),
    compiler_params=pltpu.CompilerParams(
        needs_layout_passes=False, use_tc_tiling_on_sc=False),
    scratch_types=[
        pltpu.VMEM((CH,), jnp.int32),
        pltpu.VMEM((CH,), jnp.int32),
        pltpu.VMEM((CH,), jnp.float32),
        pltpu.VMEM((N,), jnp.float32),
        pltpu.VMEM((N + 16,), jnp.float32),
        pltpu.VMEM((CH, WROW), jnp.float32),
        pltpu.VMEM_SHARED((NPAD, WROW), jnp.float32),
        pltpu.SemaphoreType.DMA,
    ],
)(_edge_body)


def _tc_a_body(x_ref, w_ref, avs_ref, avd_ref, hpad_ref, as_ref, ad_ref):
    h = jnp.dot(x_ref[...], w_ref[...], preferred_element_type=jnp.float32)
    hpad_ref[:, :D] = h
    pad = (lax.broadcasted_iota(jnp.int32, (BN, WROW - D), 1) == 0)
    hpad_ref[:, D:] = pad.astype(jnp.float32)
    as_ref[...] = jnp.dot(h, avs_ref[...], preferred_element_type=jnp.float32)
    ad_ref[...] = jnp.dot(h, avd_ref[...], preferred_element_type=jnp.float32)


def _tc_a(x, w, avs, avd):
    return pl.pallas_call(
        _tc_a_body,
        grid=(NBLK,),
        in_specs=[
            pl.BlockSpec((BN, D), lambda i: (i, 0)),
            pl.BlockSpec((D, D), lambda i: (0, 0)),
            pl.BlockSpec((D, 1), lambda i: (0, 0)),
            pl.BlockSpec((D, 1), lambda i: (0, 0)),
        ],
        out_specs=[
            pl.BlockSpec((BN, WROW), lambda i: (i, 0)),
            pl.BlockSpec((BN, 1), lambda i: (i, 0)),
            pl.BlockSpec((BN, 1), lambda i: (i, 0)),
        ],
        out_shape=[
            jax.ShapeDtypeStruct((N, WROW), jnp.float32),
            jax.ShapeDtypeStruct((N, 1), jnp.float32),
            jax.ShapeDtypeStruct((N, 1), jnp.float32),
        ],
    )(x, w, avs, avd)


def _combine(part_ref, b_ref):
    p0 = part_ref[0]
    p1 = part_ref[1]
    den = p0[:, D:D + 1] + p1[:, D:D + 1] + 1e-16
    return (p0[:, :D] + p1[:, :D]) / den + b_ref[...]


def _tc_b_body(part_ref, b_ref, w_ref, avs_ref, avd_ref,
               hpad_ref, as_ref, ad_ref):
    feats = _combine(part_ref, b_ref)
    h = jnp.dot(feats, w_ref[...], preferred_element_type=jnp.float32)
    hpad_ref[:, :D] = h
    pad = (lax.broadcasted_iota(jnp.int32, (BN, WROW - D), 1) == 0)
    hpad_ref[:, D:] = pad.astype(jnp.float32)
    as_ref[...] = jnp.dot(h, avs_ref[...], preferred_element_type=jnp.float32)
    ad_ref[...] = jnp.dot(h, avd_ref[...], preferred_element_type=jnp.float32)


def _tc_b(part, b, w, avs, avd):
    return pl.pallas_call(
        _tc_b_body,
        grid=(NBLK,),
        in_specs=[
            pl.BlockSpec((2, BN, WROW), lambda i: (0, i, 0)),
            pl.BlockSpec((1, D), lambda i: (0, 0)),
            pl.BlockSpec((D, D), lambda i: (0, 0)),
            pl.BlockSpec((D, 1), lambda i: (0, 0)),
            pl.BlockSpec((D, 1), lambda i: (0, 0)),
        ],
        out_specs=[
            pl.BlockSpec((BN, WROW), lambda i: (i, 0)),
            pl.BlockSpec((BN, 1), lambda i: (i, 0)),
            pl.BlockSpec((BN, 1), lambda i: (i, 0)),
        ],
        out_shape=[
            jax.ShapeDtypeStruct((N, WROW), jnp.float32),
            jax.ShapeDtypeStruct((N, 1), jnp.float32),
            jax.ShapeDtypeStruct((N, 1), jnp.float32),
        ],
    )(part, b, w, avs, avd)


def _tc_c_body(part_ref, b_ref, batch_ref, out_ref, sums, cnt):
    i = pl.program_id(0)

    @pl.when(i == 0)
    def _():
        sums[...] = jnp.zeros_like(sums)
        cnt[...] = jnp.zeros_like(cnt)

    feats = _combine(part_ref, b_ref)
    bblk = batch_ref[0, 0, :]
    oh = (bblk[None, :] == lax.broadcasted_iota(jnp.int32, (G, BN), 0))
    oh = oh.astype(jnp.float32)
    sums[...] += jnp.dot(oh, feats, preferred_element_type=jnp.float32)
    cnt[...] += jnp.sum(oh, axis=1, keepdims=True)

    @pl.when(i == NBLK - 1)
    def _():
        out_ref[...] = sums[...] / jnp.maximum(cnt[...], 1.0)


def _tc_c(part, b, batch3):
    return pl.pallas_call(
        _tc_c_body,
        grid=(NBLK,),
        in_specs=[
            pl.BlockSpec((2, BN, WROW), lambda i: (0, i, 0)),
            pl.BlockSpec((1, D), lambda i: (0, 0)),
            pl.BlockSpec((1, 1, BN), lambda i: (i, 0, 0)),
        ],
        out_specs=pl.BlockSpec((G, D), lambda i: (0, 0)),
        out_shape=jax.ShapeDtypeStruct((G, D), jnp.float32),
        scratch_shapes=[
            pltpu.VMEM((G, D), jnp.float32),
            pltpu.VMEM((G, 1), jnp.float32),
        ],
    )(part, b, batch3)


def kernel(x, edge_index, batch, W1, att_src1, att_dst1, b1,
           W2, att_src2, att_dst2, b2):
    srcp = jnp.concatenate(
        [edge_index[0], jnp.zeros((EPAD - E,), jnp.int32)])
    dstp = jnp.concatenate(
        [edge_index[1],
         N + jnp.arange(EPAD - E, dtype=jnp.int32) % 16])

    hpad1, as1, ad1 = _tc_a(x, W1, att_src1.reshape(D, 1),
                            att_dst1.reshape(D, 1))
    part1 = _edge_pass(hpad1, as1.reshape(N), ad1.reshape(N), srcp, dstp)
    hpad2, as2, ad2 = _tc_b(part1, b1.reshape(1, D), W2,
                            att_src2.reshape(D, 1), att_dst2.reshape(D, 1))
    part2 = _edge_pass(hpad2, as2.reshape(N), ad2.reshape(N), srcp, dstp)
    return _tc_c(part2, b2.reshape(1, D), batch.reshape(NBLK, 1, BN))


# speedup vs baseline: 1.3456x; 1.0383x over previous
import functools

import jax
import jax.numpy as jnp
from jax import lax
from jax.experimental import pallas as pl
from jax.experimental.pallas import tpu as pltpu
from jax.experimental.pallas import tpu_sc as plsc

N = 10000
E = 320000
D = 128
G = 64
WROW = 144
NPAD = 10016
NTILES = 32
EPT = 10112
EPAD = EPT * NTILES
CH = 128
NCHUNK = EPT // CH
BN = 1000
NBLK = N // BN


def _edge_body(hpad_hbm, as_hbm, ad_hbm, pk_hbm, out_hbm,
               pkb, sc, dg, w_v, as_t, ad_t, rows_v, acc_sh, sem):
    c = lax.axis_index("c")
    s = lax.axis_index("s")
    wid = s * 2 + c

    pltpu.sync_copy(as_hbm, as_t)
    pltpu.sync_copy(ad_hbm, ad_t.at[pl.ds(0, N)])
    ad_t[pl.ds(N, 16)] = jnp.zeros((16,), jnp.float32)

    for b in range(16):
        for j in range(WROW // 16):
            rows_v[b, pl.ds(j * 16, 16)] = jnp.zeros((16,), jnp.float32)

    def zero_step(k, _):
        pltpu.sync_copy(rows_v.at[pl.ds(0, 16)],
                        acc_sh.at[pl.ds(s * 640 + k * 16, 16)])
        return 0
    lax.fori_loop(0, jnp.where(s == 15, 26, 40), zero_step, 0)
    plsc.subcore_barrier()

    base_e = wid * EPT

    def chunk(i, _):
        off = base_e + i * CH
        pltpu.sync_copy(pk_hbm.at[pl.ds(off, CH)], pkb)
        for j in range(CH // 16):
            pk = pkb[pl.ds(j * 16, 16)]
            sv = jnp.bitwise_and(pk, 16383)
            dv = jnp.right_shift(pk, 14)
            sc[pl.ds(j * 16, 16)] = sv
            dg[pl.ds(j * 16, 16)] = dv
            e = plsc.load_gather(as_t, [sv]) + plsc.load_gather(ad_t, [dv])
            e = jnp.maximum(e, e * 0.2)
            w_v[pl.ds(j * 16, 16)] = jnp.exp(e)
        pltpu.async_copy(hpad_hbm.at[sc], rows_v, sem).wait()
        def scale(g, _):
            wv = w_v[pl.ds(g * 16, 16)]
            for l in range(16):
                wb = wv[l]
                b = g * 16 + l
                for j in range(WROW // 16):
                    rows_v[b, pl.ds(j * 16, 16)] = (
                        rows_v[b, pl.ds(j * 16, 16)] * wb)
            return 0
        lax.fori_loop(0, CH // 16, scale, 0)
        pltpu.sync_copy(rows_v, acc_sh.at[dg], add=True)
        return 0
    lax.fori_loop(0, NCHUNK, chunk, 0)
    plsc.subcore_barrier()

    r0 = s * 624
    pltpu.sync_copy(acc_sh.at[pl.ds(r0, 640)], out_hbm.at[c, pl.ds(r0, 640)])


_edge_pass = functools.partial(
    pl.kernel,
    out_type=jax.ShapeDtypeStruct((2, N, WROW), jnp.float32),
    mesh=plsc.VectorSubcoreMesh(core_axis_name="c", subcore_axis_name="s"),
    compiler_params=pltpu.CompilerParams(
        needs_layout_passes=False, use_tc_tiling_on_sc=False),
    scratch_types=[
        pltpu.VMEM((CH,), jnp.int32),
        pltpu.VMEM((CH,), jnp.int32),
        pltpu.VMEM((CH,), jnp.int32),
        pltpu.VMEM((CH,), jnp.float32),
        pltpu.VMEM((N,), jnp.float32),
        pltpu.VMEM((N + 16,), jnp.float32),
        pltpu.VMEM((CH, WROW), jnp.float32),
        pltpu.VMEM_SHARED((NPAD, WROW), jnp.float32),
        pltpu.SemaphoreType.DMA,
    ],
)(_edge_body)


def _tc_a_body(x_ref, w_ref, avs_ref, avd_ref, hpad_ref, as_ref, ad_ref):
    h = jnp.dot(x_ref[...], w_ref[...], preferred_element_type=jnp.float32)
    hpad_ref[:, :D] = h
    pad = (lax.broadcasted_iota(jnp.int32, (BN, WROW - D), 1) == 0)
    hpad_ref[:, D:] = pad.astype(jnp.float32)
    as_ref[...] = jnp.dot(h, avs_ref[...], preferred_element_type=jnp.float32)
    ad_ref[...] = jnp.dot(h, avd_ref[...], preferred_element_type=jnp.float32)


def _tc_a(x, w, avs, avd):
    return pl.pallas_call(
        _tc_a_body,
        grid=(NBLK,),
        in_specs=[
            pl.BlockSpec((BN, D), lambda i: (i, 0)),
            pl.BlockSpec((D, D), lambda i: (0, 0)),
            pl.BlockSpec((D, 1), lambda i: (0, 0)),
            pl.BlockSpec((D, 1), lambda i: (0, 0)),
        ],
        out_specs=[
            pl.BlockSpec((BN, WROW), lambda i: (i, 0)),
            pl.BlockSpec((BN, 1), lambda i: (i, 0)),
            pl.BlockSpec((BN, 1), lambda i: (i, 0)),
        ],
        out_shape=[
            jax.ShapeDtypeStruct((N, WROW), jnp.float32),
            jax.ShapeDtypeStruct((N, 1), jnp.float32),
            jax.ShapeDtypeStruct((N, 1), jnp.float32),
        ],
    )(x, w, avs, avd)


def _combine(part_ref, b_ref):
    p0 = part_ref[0]
    p1 = part_ref[1]
    den = p0[:, D:D + 1] + p1[:, D:D + 1] + 1e-16
    return (p0[:, :D] + p1[:, :D]) / den + b_ref[...]


def _tc_b_body(part_ref, b_ref, w_ref, avs_ref, avd_ref,
               hpad_ref, as_ref, ad_ref):
    feats = _combine(part_ref, b_ref)
    h = jnp.dot(feats, w_ref[...], preferred_element_type=jnp.float32)
    hpad_ref[:, :D] = h
    pad = (lax.broadcasted_iota(jnp.int32, (BN, WROW - D), 1) == 0)
    hpad_ref[:, D:] = pad.astype(jnp.float32)
    as_ref[...] = jnp.dot(h, avs_ref[...], preferred_element_type=jnp.float32)
    ad_ref[...] = jnp.dot(h, avd_ref[...], preferred_element_type=jnp.float32)


def _tc_b(part, b, w, avs, avd):
    return pl.pallas_call(
        _tc_b_body,
        grid=(NBLK,),
        in_specs=[
            pl.BlockSpec((2, BN, WROW), lambda i: (0, i, 0)),
            pl.BlockSpec((1, D), lambda i: (0, 0)),
            pl.BlockSpec((D, D), lambda i: (0, 0)),
            pl.BlockSpec((D, 1), lambda i: (0, 0)),
            pl.BlockSpec((D, 1), lambda i: (0, 0)),
        ],
        out_specs=[
            pl.BlockSpec((BN, WROW), lambda i: (i, 0)),
            pl.BlockSpec((BN, 1), lambda i: (i, 0)),
            pl.BlockSpec((BN, 1), lambda i: (i, 0)),
        ],
        out_shape=[
            jax.ShapeDtypeStruct((N, WROW), jnp.float32),
            jax.ShapeDtypeStruct((N, 1), jnp.float32),
            jax.ShapeDtypeStruct((N, 1), jnp.float32),
        ],
    )(part, b, w, avs, avd)


def _tc_c_body(part_ref, b_ref, batch_ref, out_ref, sums, cnt):
    i = pl.program_id(0)

    @pl.when(i == 0)
    def _():
        sums[...] = jnp.zeros_like(sums)
        cnt[...] = jnp.zeros_like(cnt)

    feats = _combine(part_ref, b_ref)
    bblk = batch_ref[0, 0, :]
    oh = (bblk[None, :] == lax.broadcasted_iota(jnp.int32, (G, BN), 0))
    oh = oh.astype(jnp.float32)
    sums[...] += jnp.dot(oh, feats, preferred_element_type=jnp.float32)
    cnt[...] += jnp.sum(oh, axis=1, keepdims=True)

    @pl.when(i == NBLK - 1)
    def _():
        out_ref[...] = sums[...] / jnp.maximum(cnt[...], 1.0)


def _tc_c(part, b, batch3):
    return pl.pallas_call(
        _tc_c_body,
        grid=(NBLK,),
        in_specs=[
            pl.BlockSpec((2, BN, WROW), lambda i: (0, i, 0)),
            pl.BlockSpec((1, D), lambda i: (0, 0)),
            pl.BlockSpec((1, 1, BN), lambda i: (i, 0, 0)),
        ],
        out_specs=pl.BlockSpec((G, D), lambda i: (0, 0)),
        out_shape=jax.ShapeDtypeStruct((G, D), jnp.float32),
        scratch_shapes=[
            pltpu.VMEM((G, D), jnp.float32),
            pltpu.VMEM((G, 1), jnp.float32),
        ],
    )(part, b, batch3)


def kernel(x, edge_index, batch, W1, att_src1, att_dst1, b1,
           W2, att_src2, att_dst2, b2):
    srcp = jnp.concatenate(
        [edge_index[0], jnp.zeros((EPAD - E,), jnp.int32)])
    dstp = jnp.concatenate(
        [edge_index[1],
         N + jnp.arange(EPAD - E, dtype=jnp.int32) % 16])
    packed = jnp.bitwise_or(srcp, jnp.left_shift(dstp, 14))

    hpad1, as1, ad1 = _tc_a(x, W1, att_src1.reshape(D, 1),
                            att_dst1.reshape(D, 1))
    part1 = _edge_pass(hpad1, as1.reshape(N), ad1.reshape(N), packed)
    hpad2, as2, ad2 = _tc_b(part1, b1.reshape(1, D), W2,
                            att_src2.reshape(D, 1), att_dst2.reshape(D, 1))
    part2 = _edge_pass(hpad2, as2.reshape(N), ad2.reshape(N), packed)
    return _tc_c(part2, b2.reshape(1, D), batch.reshape(NBLK, 1, BN))
